# Initial kernel scaffold; baseline (speedup 1.0000x reference)
import jax
import jax.numpy as jnp
from jax.experimental import pallas as pl


def kernel(x, edge_index, edge_attr, params):
    raise NotImplementedError("write your pallas kernel here")


# speedup vs baseline: 2.3825x; 2.3825x over previous
import functools

import numpy as np

import jax
import jax.numpy as jnp
from jax import lax
from jax.experimental import pallas as pl
from jax.experimental.pallas import tpu as pltpu
from jax.experimental.pallas import tpu_sc as plsc

N = 10000
E = 320000
D = 128
ED = 16
H = 128
O = 64

NC = 2
NS = 16
NW = NC * NS
CHUNK = 128
CPW = -(-E // (NW * CHUNK))
EP = NW * CPW * CHUNK
NP = 10240
NP2 = NP // 2
DUMMY = N
RPT = NP // NS
RPT2 = NP2 // NS

_f32 = jnp.float32
_i32 = jnp.int32

_sc_mesh = plsc.VectorSubcoreMesh(core_axis_name="c", subcore_axis_name="s")
_sc_params = pltpu.CompilerParams(needs_layout_passes=False)


def _zero_buf(o_v, rows, width):
    zeros16 = jnp.zeros((16,), _f32)

    def _zrow(e, _):
        for k in range(width // 16):
            o_v[e, pl.ds(k * 16, 16)] = zeros16
        return _i32(0)
    lax.fori_loop(_i32(0), _i32(rows), _zrow, _i32(0))


@functools.partial(
    pl.kernel, mesh=_sc_mesh, compiler_params=_sc_params,
    out_type=[jax.ShapeDtypeStruct((NC, NP, H), _f32)],
    scratch_types=[
        pltpu.VMEM((CHUNK,), jnp.int32),
        pltpu.VMEM((CHUNK,), jnp.int32),
        pltpu.VMEM((CHUNK, H), _f32),
        pltpu.VMEM((CHUNK, H), _f32),
        pltpu.VMEM_SHARED((NP, H), _f32),
        pltpu.SemaphoreType.DMA,
        pltpu.SemaphoreType.DMA,
    ])
def _sc_pass_c1(a_hbm, b_hbm, c_hbm, dst_hbm, src_hbm,
                out_hbm, dst_v, src_v, b_v, o_v,
                s_sh, sem_a, sem_b):
    cid = lax.axis_index("c").astype(_i32)
    sid = lax.axis_index("s").astype(_i32)
    wid = sid * _i32(NC) + cid

    _zero_buf(o_v, CHUNK, H)
    r0 = sid * _i32(RPT)
    for i in range(RPT // CHUNK):
        pltpu.sync_copy(o_v, s_sh.at[pl.ds(r0 + _i32(i * CHUNK), CHUNK)])
    plsc.subcore_barrier()

    base_w = wid * _i32(CPW * CHUNK)

    def _chunk(ch, _):
        base = base_w + ch * _i32(CHUNK)
        pltpu.sync_copy(dst_hbm.at[pl.ds(base, CHUNK)], dst_v)
        pltpu.sync_copy(src_hbm.at[pl.ds(base, CHUNK)], src_v)
        cp_a = pltpu.async_copy(a_hbm.at[dst_v], o_v, sem_a)
        cp_b = pltpu.async_copy(b_hbm.at[src_v], b_v, sem_b)
        cp_a.wait()
        cp_b.wait()

        def _add(e, _):
            for k in range(H // 16):
                sl = pl.ds(k * 16, 16)
                o_v[e, sl] = o_v[e, sl] + b_v[e, sl]
            return _i32(0)
        lax.fori_loop(_i32(0), _i32(CHUNK), _add, _i32(0))

        pltpu.sync_copy(c_hbm.at[pl.ds(base, CHUNK)], b_v)

        def _relu(e, _):
            for k in range(H // 16):
                sl = pl.ds(k * 16, 16)
                o_v[e, sl] = jnp.maximum(o_v[e, sl] + b_v[e, sl],
                                         jnp.float32(0.0))
            return _i32(0)
        lax.fori_loop(_i32(0), _i32(CHUNK), _relu, _i32(0))

        pltpu.sync_copy(o_v, s_sh.at[dst_v], add=True)
        return _i32(0)

    lax.fori_loop(_i32(0), _i32(CPW), _chunk, _i32(0))
    plsc.subcore_barrier()

    pltpu.sync_copy(s_sh.at[pl.ds(r0, RPT)],
                    out_hbm.at[cid, pl.ds(r0, RPT)])


@functools.partial(
    pl.kernel, mesh=_sc_mesh, compiler_params=_sc_params,
    out_type=[jax.ShapeDtypeStruct((NW, NP), _f32)],
    scratch_types=[
        pltpu.VMEM((CHUNK,), jnp.int32),
        pltpu.VMEM((NP,), _f32),
        pltpu.SemaphoreType.DMA,
    ])
def _sc_deg(dst_hbm, deg_hbm, dst_v, deg_v, sem):
    cid = lax.axis_index("c").astype(_i32)
    sid = lax.axis_index("s").astype(_i32)
    wid = sid * _i32(NC) + cid

    zeros16 = jnp.zeros((16,), _f32)

    def _zdeg(g, _):
        deg_v[pl.ds(g * _i32(16), 16)] = zeros16
        return _i32(0)
    lax.fori_loop(_i32(0), _i32(NP // 16), _zdeg, _i32(0))

    lane = lax.iota(jnp.int32, 16)
    masks = [lane == _i32(j) for j in range(16)]
    ones16 = jnp.ones((16,), _f32)
    base_w = wid * _i32(CPW * CHUNK)

    def _chunk(ch, _):
        base = base_w + ch * _i32(CHUNK)
        pltpu.sync_copy(dst_hbm.at[pl.ds(base, CHUNK)], dst_v)
        for g in range(CHUNK // 16):
            idx16 = dst_v[pl.ds(g * 16, 16)]
            for j in range(16):
                plsc.addupdate_scatter(deg_v, [idx16], ones16,
                                       mask=masks[j])
        return _i32(0)

    lax.fori_loop(_i32(0), _i32(CPW), _chunk, _i32(0))
    pltpu.sync_copy(deg_v, deg_hbm.at[wid])


@functools.partial(
    pl.kernel, mesh=_sc_mesh, compiler_params=_sc_params,
    out_type=[jax.ShapeDtypeStruct((NC, NP2, H), _f32)],
    scratch_types=[
        pltpu.VMEM((CHUNK,), jnp.int32),
        pltpu.VMEM((CHUNK,), jnp.int32),
        pltpu.VMEM((CHUNK,), jnp.int32),
        pltpu.VMEM((CHUNK,), _f32),
        pltpu.VMEM((CHUNK, H), _f32),
        pltpu.VMEM((CHUNK, H), _f32),
        pltpu.VMEM((CHUNK, O), _f32),
        pltpu.VMEM((CHUNK, H), _f32),
        pltpu.VMEM_SHARED((NP2, H), _f32),
        pltpu.SemaphoreType.DMA,
        pltpu.SemaphoreType.DMA,
        pltpu.SemaphoreType.DMA,
    ])
def _sc_pass_c2(t_hbm, c_hbm, dst_hbm, src_hbm,
                out_hbm, dst_v, src_v, hix_v, par_v, a_v, b_v, c_v, o_v,
                s_sh, sem_a, sem_b, sem_c):
    cid = lax.axis_index("c").astype(_i32)
    sid = lax.axis_index("s").astype(_i32)
    wid = sid * _i32(NC) + cid

    _zero_buf(o_v, CHUNK, H)
    r0 = sid * _i32(RPT2)
    for off in (0, 128, 192):
        pltpu.sync_copy(o_v, s_sh.at[pl.ds(r0 + _i32(off), CHUNK)])
    plsc.subcore_barrier()

    one16 = jnp.ones((16,), _f32)
    base_w = wid * _i32(CPW * CHUNK)

    def _chunk(ch, _):
        base = base_w + ch * _i32(CHUNK)
        pltpu.sync_copy(dst_hbm.at[pl.ds(base, CHUNK)], dst_v)
        pltpu.sync_copy(src_hbm.at[pl.ds(base, CHUNK)], src_v)
        for g in range(CHUNK // 16):
            sl = pl.ds(g * 16, 16)
            d16 = dst_v[sl]
            hix_v[sl] = lax.shift_right_logical(d16, _i32(1))
            par_v[sl] = lax.convert_element_type(
                lax.bitwise_and(d16, _i32(1)), _f32)
        cp_a = pltpu.async_copy(t_hbm.at[dst_v], a_v, sem_a)
        cp_b = pltpu.async_copy(t_hbm.at[src_v], b_v, sem_b)
        cp_c = pltpu.async_copy(c_hbm.at[pl.ds(base, CHUNK)], c_v, sem_c)
        cp_a.wait()
        cp_b.wait()
        cp_c.wait()

        def _edge(e, _):
            ev = jnp.full((16,), e, _i32)
            par = plsc.load_gather(par_v, [ev])
            npar = one16 - par
            for k in range(O // 16):
                r = jnp.maximum(
                    a_v[e, pl.ds(k * 16, 16)]
                    + b_v[e, pl.ds(O + k * 16, 16)]
                    + c_v[e, pl.ds(k * 16, 16)],
                    jnp.float32(0.0))
                o_v[e, pl.ds(k * 16, 16)] = r * npar
                o_v[e, pl.ds(O + k * 16, 16)] = r * par
            return _i32(0)
        lax.fori_loop(_i32(0), _i32(CHUNK), _edge, _i32(0))

        pltpu.sync_copy(o_v, s_sh.at[hix_v], add=True)
        return _i32(0)

    lax.fori_loop(_i32(0), _i32(CPW), _chunk, _i32(0))
    plsc.subcore_barrier()

    pltpu.sync_copy(s_sh.at[pl.ds(r0, RPT2)],
                    out_hbm.at[cid, pl.ds(r0, RPT2)])


_BN = 1024
_BE = 2048


_Z = np.int32(0)


def _row_spec(bn, w):
    return pl.BlockSpec((bn, w), lambda i: (i, _Z))


def _full_spec(shape):
    return pl.BlockSpec(shape, lambda i, _n=len(shape): (_Z,) * _n)


def _tc_pre_body(x_ref, g_ref, bl_ref, pw_ref, pb_ref, wa_ref, wb_ref,
                 h_ref, a_ref, b_ref):
    xx = x_ref[...]
    mu = jnp.mean(xx, axis=1, keepdims=True)
    xc = xx - mu
    var = jnp.mean(xc * xc, axis=1, keepdims=True)
    hh = xc * lax.rsqrt(var + 1e-5) * g_ref[...] + bl_ref[...]
    hh = jnp.maximum(jnp.dot(hh, pw_ref[...],
                             preferred_element_type=_f32) + pb_ref[...], 0.0)
    h_ref[...] = hh
    a_ref[...] = jnp.dot(hh, wa_ref[...], preferred_element_type=_f32)
    b_ref[...] = jnp.dot(hh, wb_ref[...], preferred_element_type=_f32)


def _tc_pre(x_p, ln_g, ln_b, proj_W, proj_b, w1a, w1b):
    return pl.pallas_call(
        _tc_pre_body,
        grid=(NP // _BN,),
        in_specs=[
            _row_spec(_BN, D),
            _full_spec((1, D)), _full_spec((1, D)),
            _full_spec((D, H)), _full_spec((1, H)),
            _full_spec((H, H)), _full_spec((H, H)),
        ],
        out_specs=[_row_spec(_BN, H)] * 3,
        out_shape=[jax.ShapeDtypeStruct((NP, H), _f32)] * 3,
    )(x_p, ln_g, ln_b, proj_W, proj_b, w1a, w1b)


def _tc_edge_body(ea_ref, w1_ref, b1_ref, w2_ref, b2_ref, c1_ref, c2_ref):
    ea = ea_ref[...]
    dn = (((0,), (0,)), ((), ()))
    c1_ref[...] = lax.dot_general(ea, w1_ref[...], dn,
                                  preferred_element_type=_f32) + b1_ref[...]
    c2_ref[...] = lax.dot_general(ea, w2_ref[...], dn,
                                  preferred_element_type=_f32) + b2_ref[...]


def _tc_edge(ea_p, w1c, b1, w2c, b2):
    return pl.pallas_call(
        _tc_edge_body,
        grid=(EP // _BE,),
        in_specs=[
            pl.BlockSpec((ED, _BE), lambda i: (_Z, i)),
            _full_spec((ED, H)), _full_spec((1, H)),
            _full_spec((ED, O)), _full_spec((1, O)),
        ],
        out_specs=[_row_spec(_BE, H), _row_spec(_BE, O)],
        out_shape=[jax.ShapeDtypeStruct((EP, H), _f32),
                   jax.ShapeDtypeStruct((EP, O), _f32)],
    )(ea_p, w1c, b1, w2c, b2)


def _tc_deg_body(d_ref, out_ref):
    out_ref[...] = jnp.broadcast_to(jnp.sum(d_ref[...], axis=0)[:, None],
                                    (NP, 128))


def _tc_deg(degp):
    return pl.pallas_call(
        _tc_deg_body,
        grid=(1,),
        in_specs=[_full_spec((NW, NP))],
        out_specs=[_full_spec((NP, 128))],
        out_shape=[jax.ShapeDtypeStruct((NP, 128), _f32)],
    )(degp)[0]


def _tc_mid_body(h_ref, sa_ref, sb_ref, d_ref, w2_ref, b2_ref, u1t_ref,
                 u1b_ref, ub1_ref, u2_ref, ub2_ref, g_ref, bb_ref,
                 wab_ref, hn_ref, t_ref):
    f = sa_ref[...] + sb_ref[...]
    cnt = d_ref[...]
    deg = jnp.maximum(cnt, 1.0)
    agg = jnp.dot(f / deg, w2_ref[...], preferred_element_type=_f32)
    agg = agg + jnp.where(cnt > 0, jnp.float32(1.0), jnp.float32(0.0)) * b2_ref[...]
    hh = h_ref[...]
    u = jnp.maximum(jnp.dot(hh, u1t_ref[...], preferred_element_type=_f32)
                    + jnp.dot(agg, u1b_ref[...], preferred_element_type=_f32)
                    + ub1_ref[...], 0.0)
    u = jnp.dot(u, u2_ref[...], preferred_element_type=_f32) + ub2_ref[...]
    u = u * jnp.float32(1.0 / (1.0 + 1e-5) ** 0.5) * g_ref[...] + bb_ref[...]
    hn = hh + jnp.maximum(u, 0.0)
    hn_ref[...] = hn
    t_ref[...] = jnp.dot(hn, wab_ref[...], preferred_element_type=_f32)


def _tc_mid(h, s1a, s1b, cnt1, w2, b2, u1t, u1b, ub1, u2, ub2, g, bb, w2ab):
    return pl.pallas_call(
        _tc_mid_body,
        grid=(NP // _BN,),
        in_specs=[
            _row_spec(_BN, H),
            _row_spec(_BN, H),
            _row_spec(_BN, H),
            _row_spec(_BN, 128),
            _full_spec((H, H)), _full_spec((1, H)),
            _full_spec((H, H)), _full_spec((H, H)), _full_spec((1, H)),
            _full_spec((H, H)), _full_spec((1, H)),
            _full_spec((1, H)), _full_spec((1, H)),
            _full_spec((H, H)),
        ],
        out_specs=[_row_spec(_BN, H), _row_spec(_BN, H)],
        out_shape=[jax.ShapeDtypeStruct((NP, H), _f32),
                   jax.ShapeDtypeStruct((NP, H), _f32)],
    )(h, s1a, s1b, cnt1, w2, b2, u1t, u1b, ub1, u2, ub2, g, bb, w2ab)


def _tc_post_body(h_ref, sa_ref, sb_ref, d_ref, w2_ref, b2_ref, u1t_ref,
                  u1b_ref, ub1_ref, u2_ref, ub2_ref, g_ref, bb_ref,
                  rw_ref, rb_ref,
                  cw1_ref, cb1_ref, cw2_ref, cb2_ref, out_ref):
    s = sa_ref[...] + sb_ref[...]
    cnt = d_ref[:, :O]
    deg = jnp.maximum(cnt, 1.0)
    agg = jnp.dot(s / deg, w2_ref[...], preferred_element_type=_f32)
    agg = agg + jnp.where(cnt > 0, jnp.float32(1.0), jnp.float32(0.0)) * b2_ref[...]
    hh = h_ref[...]
    u = jnp.maximum(jnp.dot(hh, u1t_ref[...], preferred_element_type=_f32)
                    + jnp.dot(agg, u1b_ref[...], preferred_element_type=_f32)
                    + ub1_ref[...], 0.0)
    u = jnp.dot(u, u2_ref[...], preferred_element_type=_f32) + ub2_ref[...]
    u = u * jnp.float32(1.0 / (1.0 + 1e-5) ** 0.5) * g_ref[...] + bb_ref[...]
    h2 = jnp.maximum(u, 0.0)
    ho = jnp.dot(hh, rw_ref[...], preferred_element_type=_f32) + rb_ref[...] + h2
    z = jnp.maximum(jnp.dot(ho, cw1_ref[...],
                            preferred_element_type=_f32) + cb1_ref[...], 0.0)
    lo = jnp.sum(z * cw2_ref[...], axis=1, keepdims=True) + cb2_ref[:, 0:1]
    out_ref[...] = jnp.broadcast_to(lo, (_BN, 128))


def _tc_post(h, s2a, s2b, cnt1, w2, b2, u1t, u1b, ub1, u2, ub2, g, bb,
             rw, rb, cw1, cb1, cw2, cb2):
    return pl.pallas_call(
        _tc_post_body,
        grid=(NP // _BN,),
        in_specs=[
            _row_spec(_BN, H),
            _row_spec(_BN, O),
            _row_spec(_BN, O),
            _row_spec(_BN, 128),
            _full_spec((O, O)), _full_spec((1, O)),
            _full_spec((H, O)), _full_spec((O, O)), _full_spec((1, O)),
            _full_spec((O, O)), _full_spec((1, O)),
            _full_spec((1, O)), _full_spec((1, O)),
            _full_spec((H, O)), _full_spec((1, O)),
            _full_spec((O, O // 2)), _full_spec((1, O // 2)),
            _full_spec((1, O // 2)), _full_spec((1, 128)),
        ],
        out_specs=[_row_spec(_BN, 128)],
        out_shape=[jax.ShapeDtypeStruct((NP, 128), _f32)],
    )(h, s2a, s2b, cnt1, w2, b2, u1t, u1b, ub1, u2, ub2, g, bb,
      rw, rb, cw1, cb1, cw2, cb2)[0]


def kernel(x, edge_index, edge_attr, params):
    p = params
    x = x.astype(_f32)
    edge_attr = edge_attr.astype(_f32)
    src = edge_index[0].astype(jnp.int32)
    dst = edge_index[1].astype(jnp.int32)

    pad_e = EP - E
    src_p = jnp.concatenate([src, jnp.full((pad_e,), DUMMY, jnp.int32)])
    dst_p = jnp.concatenate([dst, jnp.full((pad_e,), DUMMY, jnp.int32)])
    ea_p = jnp.concatenate([edge_attr.T, jnp.zeros((ED, pad_e), _f32)],
                           axis=1)
    x_p = jnp.concatenate([x, jnp.zeros((NP - N, D), _f32)])

    def row(v):
        return v.reshape(1, -1).astype(_f32)

    w1 = p['c1_msg_W1']
    h, a1, b1 = _tc_pre(x_p, row(p['ln_g']), row(p['ln_b']),
                        p['proj_W'], row(p['proj_b']),
                        w1[:H], w1[H:2 * H])
    w1c2 = p['c2_msg_W1']
    c1, c2 = _tc_edge(ea_p, w1[2 * H:], row(p['c1_msg_b1']),
                      w1c2[2 * H:], row(p['c2_msg_b1']))

    (degp,) = _sc_deg(dst_p)
    cnt1 = _tc_deg(degp)
    (s1p,) = _sc_pass_c1(a1, b1, c1, dst_p, src_p)

    w2ab = jnp.concatenate([w1c2[:H], w1c2[H:2 * H]], axis=1)
    hn, t2 = _tc_mid(h, s1p[0], s1p[1], cnt1,
                     p['c1_msg_W2'], row(p['c1_msg_b2']),
                     p['c1_upd_W1'][:H], p['c1_upd_W1'][H:],
                     row(p['c1_upd_b1']), p['c1_upd_W2'],
                     row(p['c1_upd_b2']), row(p['bn1_g']),
                     row(p['bn1_b']), w2ab)

    (s2pk,) = _sc_pass_c2(t2, c2, dst_p, src_p)
    s2 = s2pk.reshape(NC, NP, O)

    out = _tc_post(hn, s2[0], s2[1], cnt1,
                   p['c2_msg_W2'], row(p['c2_msg_b2']),
                   p['c2_upd_W1'][:H], p['c2_upd_W1'][H:],
                   row(p['c2_upd_b1']), p['c2_upd_W2'], row(p['c2_upd_b2']),
                   row(p['bn2_g']), row(p['bn2_b']),
                   p['res2_W'], row(p['res2_b']),
                   p['cls_W1'], row(p['cls_b1']),
                   p['cls_W2'].reshape(1, O // 2),
                   jnp.tile(p['cls_b2'].reshape(1, 1), (1, 128)))
    return out[:N, 0]
